# baseline (speedup 1.0000x reference)
import functools

import jax
import jax.numpy as jnp
from jax import lax
from jax.experimental import pallas as pl
from jax.experimental.pallas import tpu as pltpu
from jax.experimental.pallas import tpu_sc as plsc

D = 64
SCALE = 8.0
NBUF = 4
BB = 128
VB = 128

_PARAMS = dict(use_tc_tiling_on_sc=True, needs_layout_passes=False)


@functools.lru_cache(maxsize=None)
def _make_repack_kernel(vocab: int):
    info = plsc.get_sparse_core_info()
    nw = info.num_cores * info.num_subcores
    nblk = vocab // VB
    vrem = vocab - nblk * VB
    npw = (nblk + nw - 1) // nw
    npw += npw % 2
    npairs = npw // 2
    mesh = plsc.VectorSubcoreMesh(core_axis_name="c", subcore_axis_name="s")
    iota = None

    @functools.partial(
        pl.kernel,
        mesh=mesh,
        out_type=jax.ShapeDtypeStruct((vocab // 2, 2 * D), jnp.float32),
        compiler_params=pltpu.CompilerParams(**_PARAMS),
        scratch_types=[
            [pltpu.VMEM((D, VB), jnp.float32) for _ in range(2)],
            [pltpu.VMEM((VB // 2, 2 * D), jnp.float32) for _ in range(2)],
            [pltpu.SemaphoreType.DMA for _ in range(2)],
            [pltpu.SemaphoreType.DMA for _ in range(2)],
        ],
    )
    def k(tt_hbm, rem_hbm, t2_hbm, srcs, outs, isems, osems):
        wid = lax.axis_index("s") * info.num_cores + lax.axis_index("c")
        iota = lax.iota(jnp.int32, 16)

        def src_copy(blk, b):
            src = tt_hbm.at[:, pl.ds(blk * VB, VB)]
            return pltpu.make_async_copy(src, srcs[b], isems[b])

        def out_copy(blk, b):
            dst = t2_hbm.at[pl.ds(blk * (VB // 2), VB // 2)]
            return pltpu.make_async_copy(outs[b], dst, osems[b])

        perms = [lax.bitwise_and(iota + kk, 15) for kk in range(16)]

        def transform(b, nv):
            s, o = srcs[b], outs[b]
            for d0 in range(D // 16):
                dd = d0 * 16 + iota

                @plsc.parallel_loop(0, nv // 16, unroll=1)
                def _(vt):
                    vbase = vt * 16
                    for kk in range(16):
                        vv = vbase + perms[kk]
                        row = lax.shift_right_logical(vv, 1)
                        col = lax.shift_left(lax.bitwise_and(vv, 1), 6) + dd
                        v = plsc.load_gather(s, [dd, vv])
                        plsc.store_scatter(o, [row, col], v * SCALE)

        def blk_of(j):
            return lax.rem(wid + j * nw, nblk)

        src_copy(blk_of(0), 0).start()

        def body(p, carry):
            for b in range(2):
                j = 2 * p + b
                src_copy(blk_of(j + 1), 1 - b).start()
                src_copy(blk_of(j), b).wait()

                @pl.when(j >= 2)
                def _():
                    out_copy(blk_of(j - 2), b).wait()

                transform(b, VB)
                out_copy(blk_of(j), b).start()
            return carry

        lax.fori_loop(0, npairs - 1, body, 0)

        for b in range(2):
            j = npw - 2 + b
            if j + 1 < npw:
                src_copy(blk_of(j + 1), 1 - b).start()
            src_copy(blk_of(j), b).wait()
            out_copy(blk_of(j - 2), b).wait()
            transform(b, VB)
            out_copy(blk_of(j), b).start()
        for b in range(2):
            out_copy(blk_of(npw - 2 + b), b).wait()

        if vrem:
            @pl.when(wid == 0)
            def _():
                stage = srcs[0].at[pl.ds(0, vrem // 2), :]
                pltpu.make_async_copy(rem_hbm, stage, isems[0]).start()
                pltpu.make_async_copy(rem_hbm, stage, isems[0]).wait()
                dst = t2_hbm.at[pl.ds(nblk * (VB // 2), vrem // 2)]
                pltpu.make_async_copy(stage, dst, osems[0]).start()
                pltpu.make_async_copy(stage, dst, osems[0]).wait()

    return k


@functools.lru_cache(maxsize=None)
def _make_gather_kernel(nbatch: int, seq: int, vocab: int):
    info = plsc.get_sparse_core_info()
    nw = info.num_cores * info.num_subcores
    n_units = seq * (nbatch // BB)
    u_per_w = n_units // nw
    nquads = u_per_w // NBUF
    ncols = nbatch // BB
    mesh = plsc.VectorSubcoreMesh(core_axis_name="c", subcore_axis_name="s")

    @functools.partial(
        pl.kernel,
        mesh=mesh,
        out_type=jax.ShapeDtypeStruct((seq, D // 8, ncols, 8, BB), jnp.float32),
        compiler_params=pltpu.CompilerParams(**_PARAMS),
        scratch_types=[
            [pltpu.VMEM((BB,), jnp.int32) for _ in range(NBUF)],
            [pltpu.VMEM((BB,), jnp.int32) for _ in range(NBUF)],
            [pltpu.VMEM((BB,), jnp.int32) for _ in range(NBUF)],
            [pltpu.VMEM((BB, 2 * D), jnp.float32) for _ in range(NBUF)],
            [pltpu.VMEM((D // 8, 8, BB), jnp.float32) for _ in range(NBUF)],
            [pltpu.SemaphoreType.DMA for _ in range(NBUF)],
            [pltpu.SemaphoreType.DMA for _ in range(NBUF)],
            [pltpu.SemaphoreType.DMA for _ in range(NBUF)],
        ],
    )
    def k(idx_hbm, t2_hbm, out_hbm, idxs, idx2s, pars, rows, outs,
          isems, gsems, osems):
        wid = lax.axis_index("s") * info.num_cores + lax.axis_index("c")
        ubase = wid * u_per_w
        iota = lax.iota(jnp.int32, 16)

        def unit_sc(u):
            g = ubase + u
            return lax.div(g, ncols), lax.rem(g, ncols)

        def idx_copy(u, b):
            s, c = unit_sc(u)
            src = idx_hbm.at[s, pl.ds(c * BB, BB)]
            return pltpu.make_async_copy(src, idxs[b], isems[b])

        def gather_copy(b):
            return pltpu.make_async_copy(t2_hbm.at[idx2s[b]], rows[b], gsems[b])

        def out_copy(u, b):
            s, c = unit_sc(u)
            dst = out_hbm.at[s, :, c, :, :]
            return pltpu.make_async_copy(outs[b], dst, osems[b])

        def shift_compute(b):
            def cb(c0, car):
                sl = pl.ds(c0 * 16, 16)
                v = idxs[b][sl]
                idx2s[b][sl] = lax.shift_right_logical(v, 1)
                pars[b][sl] = lax.shift_left(lax.bitwise_and(v, 1), 6)
                return car

            lax.fori_loop(0, BB // 16, cb, 0)

        perms = [lax.bitwise_and(iota + kk, 15) for kk in range(16)]

        def transform(b):
            rows_b, par_b, outb = rows[b], pars[b], outs[b]

            @plsc.parallel_loop(0, BB // 16, unroll=1)
            def _(ct):
                sl = pl.ds(ct * 16, 16)
                par = par_b[sl]
                rowi = ct * 16 + iota
                for d0 in range(D // 16):
                    for kk in range(16):
                        dd = perms[kk] + d0 * 16
                        v = plsc.load_gather(rows_b, [rowi, par + dd])
                        rv = lax.shift_right_logical(dd, 3)
                        rrv = lax.bitwise_and(dd, 7)
                        plsc.store_scatter(outb, [rv, rrv, rowi], v)

        for b in range(NBUF):
            idx_copy(b, b).start()
        for b in range(NBUF):
            idx_copy(b, b).wait()
            shift_compute(b)
            gather_copy(b).start()

        def body(q, carry):
            u0 = q * NBUF
            for b in range(NBUF):
                u1 = u0 + NBUF + b

                @pl.when(u1 < u_per_w)
                def _():
                    idx_copy(u1, b).start()

            for b in range(NBUF):
                u = u0 + b

                @pl.when(q > 0)
                def _():
                    out_copy(u - NBUF, b).wait()

                gather_copy(b).wait()
                transform(b)
                out_copy(u, b).start()
            for b in range(NBUF):
                u1 = u0 + NBUF + b

                @pl.when(u1 < u_per_w)
                def _():
                    idx_copy(u1, b).wait()
                    shift_compute(b)
                    gather_copy(b).start()

            return carry

        lax.fori_loop(0, nquads, body, 0)

        u0 = (nquads - 1) * NBUF
        for b in range(NBUF):
            out_copy(u0 + b, b).wait()

    return k


def kernel(xb, table):
    nb, seq = xb.shape
    vocab = table.shape[0]
    xbT = xb.T.astype(jnp.int32)
    nblk = vocab // VB
    rem = (table[nblk * VB:] * SCALE).reshape(-1, 2 * D)
    t2 = _make_repack_kernel(vocab)(table.T, rem)
    a = _make_gather_kernel(nb, seq, vocab)(xbT, t2)
    return a.transpose((2, 4, 0, 1, 3)).reshape(nb, seq, D)


# speedup vs baseline: 1.4092x; 1.4092x over previous
import functools

import jax
import jax.numpy as jnp
from jax import lax
from jax.experimental import pallas as pl
from jax.experimental.pallas import tpu as pltpu
from jax.experimental.pallas import tpu_sc as plsc

D = 64
SCALE = 8.0
NBUF = 4
BB = 128


@functools.lru_cache(maxsize=None)
def _make_sc_kernel(nbatch: int, seq: int, vocab: int):
    info = plsc.get_sparse_core_info()
    nw = info.num_cores * info.num_subcores
    n_units = seq * (nbatch // BB)
    u_per_w = n_units // nw
    nquads = u_per_w // NBUF
    ncols = nbatch // BB
    mesh = plsc.VectorSubcoreMesh(core_axis_name="c", subcore_axis_name="s")

    @functools.partial(
        pl.kernel,
        mesh=mesh,
        out_type=jax.ShapeDtypeStruct((seq, D // 8, ncols, 8, BB), jnp.float32),
        compiler_params=pltpu.CompilerParams(
            use_tc_tiling_on_sc=False, needs_layout_passes=False
        ),
        scratch_types=[
            [pltpu.VMEM((BB,), jnp.int32) for _ in range(NBUF)],
            [pltpu.VMEM((BB, D), jnp.float32) for _ in range(NBUF)],
            [pltpu.VMEM((D // 8, 8, BB + 8), jnp.float32) for _ in range(NBUF)],
            [pltpu.SemaphoreType.DMA for _ in range(NBUF)],
            [pltpu.SemaphoreType.DMA for _ in range(NBUF)],
            [pltpu.SemaphoreType.DMA for _ in range(NBUF)],
        ],
    )
    def k(idx_hbm, t2_hbm, out_hbm, idxs, rows, outs, isems, gsems, osems):
        wid = lax.axis_index("s") * info.num_cores + lax.axis_index("c")
        ubase = wid * u_per_w

        def unit_sc(u):
            g = ubase + u
            return lax.div(g, ncols), lax.rem(g, ncols)

        def idx_copy(u, b):
            s, c = unit_sc(u)
            src = idx_hbm.at[s, pl.ds(c * BB, BB)]
            return pltpu.make_async_copy(src, idxs[b], isems[b])

        def gather_copy(b):
            return pltpu.make_async_copy(t2_hbm.at[idxs[b]], rows[b], gsems[b])

        def out_copy(u, b):
            s, c = unit_sc(u)
            src = outs[b].at[:, :, pl.ds(0, BB)]
            dst = out_hbm.at[s, :, c, :, :]
            return pltpu.make_async_copy(src, dst, osems[b])

        def transform(b):
            rows_b, outb = rows[b], outs[b]
            iota = lax.iota(jnp.int32, 16)

            def d_body(t, car):
                dv = t * 16 + iota
                rv = lax.shift_right_logical(dv, 3)
                rrv = lax.bitwise_and(dv, 7)

                @plsc.parallel_loop(0, BB, unroll=4)
                def _(c):
                    v = rows_b[c, pl.ds(t * 16, 16)] * SCALE
                    cv = jnp.full((16,), c, jnp.int32)
                    plsc.store_scatter(outb, [rv, rrv, cv], v)

                return car

            lax.fori_loop(0, D // 16, d_body, 0)

        for b in range(NBUF):
            idx_copy(b, b).start()
        for b in range(NBUF):
            idx_copy(b, b).wait()
            gather_copy(b).start()

        def body(q, carry):
            u0 = q * NBUF
            for b in range(NBUF):
                u1 = u0 + NBUF + b

                @pl.when(u1 < u_per_w)
                def _():
                    idx_copy(u1, b).start()

            for b in range(NBUF):
                u = u0 + b

                @pl.when(q > 0)
                def _():
                    out_copy(u - NBUF, b).wait()

                gather_copy(b).wait()
                transform(b)
                out_copy(u, b).start()
            for b in range(NBUF):
                u1 = u0 + NBUF + b

                @pl.when(u1 < u_per_w)
                def _():
                    idx_copy(u1, b).wait()
                    gather_copy(b).start()

            return carry

        lax.fori_loop(0, nquads, body, 0)

        u0 = (nquads - 1) * NBUF
        for b in range(NBUF):
            out_copy(u0 + b, b).wait()

    return k


def kernel(xb, table):
    nb, seq = xb.shape
    vocab = table.shape[0]
    xbT = xb.T.astype(jnp.int32)
    a = _make_sc_kernel(nb, seq, vocab)(xbT, table)
    return a.transpose((2, 4, 0, 1, 3)).reshape(nb, seq, D)
